# baseline (speedup 1.0000x reference)
import jax
import jax.numpy as jnp
from jax.experimental import pallas as pl

N = 10000
A = 0.5
V = 1.0

BM = 400
BM_WIDE = 1000
BM_CAST = 400

_bf = jnp.bfloat16
_f32 = jnp.float32


def _dot(a, b):
    return jnp.dot(a, b, preferred_element_type=_f32)


def _row(c):
    return pl.BlockSpec((BM, c), lambda i: (i, 0))


def _full(arr):
    return pl.BlockSpec(arr.shape, lambda i: (0,) * arr.ndim)


def _ae_body(x_ref,
             we1, be1, we2, be2, we3, be3, wz, bz,
             wd1, bd1, wd2, bd2, wd3, bd3, wxb, bxb, wg1,
             h1_ref, h2_ref, h3_ref, zae_ref, xbar_ref, m1_ref):
    xb = x_ref[...].astype(_bf)
    h1 = jax.nn.relu(_dot(xb, we1[...]) + be1[...])
    h1_ref[...] = h1.astype(_bf)
    h2 = jax.nn.relu(_dot(h1.astype(_bf), we2[...]) + be2[...])
    h2_ref[...] = h2.astype(_bf)
    h3 = jax.nn.relu(_dot(h2.astype(_bf), we3[...]) + be3[...])
    h3_ref[...] = h3.astype(_bf)
    zae = _dot(h3.astype(_bf), wz[...]) + bz[...]
    zae_ref[...] = zae
    dd1 = jax.nn.relu(_dot(zae.astype(_bf), wd1[...]) + bd1[...])
    dd2 = jax.nn.relu(_dot(dd1.astype(_bf), wd2[...]) + bd2[...])
    dd3 = jax.nn.relu(_dot(dd2.astype(_bf), wd3[...]) + bd3[...])
    xbar_ref[...] = _dot(dd3.astype(_bf), wxb[...]) + bxb[...]
    m1_ref[...] = _dot(xb, wg1[...]).astype(_bf)


def _run_ae(x, wb, bb, wg1b):
    in_specs = [_row(512)]
    args = [x]
    for w, b in zip(wb, bb):
        in_specs += [_full(w), _full(b)]
        args += [w, b]
    in_specs.append(_full(wg1b))
    args.append(wg1b)
    out_shape = [
        jax.ShapeDtypeStruct((N, 128), _bf),
        jax.ShapeDtypeStruct((N, 256), _bf),
        jax.ShapeDtypeStruct((N, 512), _bf),
        jax.ShapeDtypeStruct((N, 64), _f32),
        jax.ShapeDtypeStruct((N, 512), _f32),
        jax.ShapeDtypeStruct((N, 128), _bf),
    ]
    out_specs = [_row(128), _row(256), _row(512), _row(64), _row(512),
                 _row(128)]
    return pl.pallas_call(
        _ae_body, grid=(N // BM,), in_specs=in_specs, out_specs=out_specs,
        out_shape=out_shape)(*args)


def _cast_l1_body(adj_ref, m1_ref, h1_ref, adjb_ref, m2_ref):
    ab = adj_ref[...].astype(_bf)
    adjb_ref[...] = ab
    ge1 = jax.nn.relu(_dot(ab, m1_ref[...]))
    m2_ref[...] = ((1.0 - A) * ge1
                   + A * h1_ref[...].astype(_f32)).astype(_bf)


def _run_cast_l1(adj, m1b, h1b):
    return pl.pallas_call(
        _cast_l1_body, grid=(N // BM_CAST,),
        in_specs=[
            pl.BlockSpec((BM_CAST, N), lambda i: (i, 0)),
            pl.BlockSpec((N, 128), lambda i: (0, 0)),
            pl.BlockSpec((BM_CAST, 128), lambda i: (i, 0)),
        ],
        out_specs=[
            pl.BlockSpec((BM_CAST, N), lambda i: (i, 0)),
            pl.BlockSpec((BM_CAST, 128), lambda i: (i, 0)),
        ],
        out_shape=[
            jax.ShapeDtypeStruct((N, N), _bf),
            jax.ShapeDtypeStruct((N, 128), _bf),
        ])(adj, m1b, h1b)


def _l2_body(adj_ref, m_ref, w_ref, h_ref, out_ref):
    acc = _dot(adj_ref[...], m_ref[...]).astype(_bf)
    ge = jax.nn.relu(_dot(acc, w_ref[...]))
    out_ref[...] = ((1.0 - A) * ge + A * h_ref[...].astype(_f32)).astype(_bf)


def _l3_body(adj_ref, m_ref, w_ref, h_ref, w2_ref, out_ref):
    acc = _dot(adj_ref[...], m_ref[...]).astype(_bf)
    ge = jax.nn.relu(_dot(acc, w_ref[...]))
    mix = ((1.0 - A) * ge + A * h_ref[...].astype(_f32)).astype(_bf)
    out_ref[...] = _dot(mix, w2_ref[...]).astype(_bf)


def _l4_body(adj_ref, m_ref, zae_ref, zgb_ref, m5_ref):
    zg = _dot(adj_ref[...], m_ref[...])
    zgb = zg.astype(_bf)
    zgb_ref[...] = zgb
    zi = ((1.0 - A) * zg + A * zae_ref[...]).astype(_bf)
    m5_ref[...] = jnp.concatenate([zi, zgb], axis=1)


def _l5_body(adj_ref, m_ref, w5_ref, w6_ref, zl_ref, m6_ref):
    acc = _dot(adj_ref[...], m_ref[...])
    zl_ref[...] = acc[:, :64]
    gd1 = jax.nn.relu(_dot(acc[:, 64:].astype(_bf), w5_ref[...])).astype(_bf)
    m6_ref[...] = _dot(gd1, w6_ref[...]).astype(_bf)


def _l6_body(adj_ref, m_ref, w_ref, out_ref):
    gd2 = jax.nn.relu(_dot(adj_ref[...], m_ref[...])).astype(_bf)
    out_ref[...] = _dot(gd2, w_ref[...]).astype(_bf)


def _l7_body(adj_ref, m_ref, zgb_ref, zgfull_ref, out_ref, ah_ref):
    out_ref[...] = jax.nn.relu(_dot(adj_ref[...], m_ref[...])).astype(_bf)
    prod = jax.lax.dot_general(
        zgb_ref[...], zgfull_ref[...], (((1,), (1,)), ((), ())),
        preferred_element_type=_f32)
    ah_ref[...] = jax.nn.sigmoid(prod)


def _softq(z, c, cc, q_ref):
    zz = jnp.sum(z * z, axis=1, keepdims=True)
    zc = jax.lax.dot_general(z, c, (((1,), (1,)), ((), ())),
                             preferred_element_type=_f32,
                             precision=jax.lax.Precision.HIGHEST)
    d2 = zz + cc - 2.0 * zc
    q = 1.0 / (1.0 + d2 / V)
    q = q ** ((V + 1.0) / 2.0)
    q_ref[...] = q / jnp.sum(q, axis=1, keepdims=True)


def _l8_body(adj_ref, m_ref, w_ref, zl_ref, zae_ref, c_ref,
             out_ref, q_ref, q1_ref):
    acc = _dot(adj_ref[...], m_ref[...]).astype(_bf)
    out_ref[...] = jax.nn.relu(_dot(acc, w_ref[...]))
    c = c_ref[...]
    cc = jnp.sum(c * c, axis=1)[None, :]
    _softq(zl_ref[...], c, cc, q_ref)
    _softq(zae_ref[...], c, cc, q1_ref)


def _spmm(body, adjb, m, extras, outs, bm=BM):
    row = lambda c: pl.BlockSpec((bm, c), lambda i: (i, 0))
    in_specs = [row(N), _full(m)]
    args = [adjb, m]
    for e, blocked in extras:
        in_specs.append(row(e.shape[1]) if blocked else _full(e))
        args.append(e)
    out_specs = [row(c) for c, _ in outs]
    out_shape = [jax.ShapeDtypeStruct((N, c), dt) for c, dt in outs]
    if len(outs) == 1:
        out_specs, out_shape = out_specs[0], out_shape[0]
    return pl.pallas_call(
        body, grid=(pl.cdiv(N, bm),), in_specs=in_specs,
        out_specs=out_specs, out_shape=out_shape)(*args)


def kernel(x, adj, We1, be1, We2, be2, We3, be3, Wz, bz, Wd1, bd1, Wd2, bd2,
           Wd3, bd3, Wxb, bxb, Wg1, Wg2, Wg3, Wg4, Wg5, Wg6, Wg7, Wg8,
           cluster):
    wb = [w.astype(_bf) for w in (We1, We2, We3, Wz, Wd1, Wd2, Wd3, Wxb)]
    bb = [b.reshape(1, -1) for b in (be1, be2, be3, bz, bd1, bd2, bd3, bxb)]
    wg = [w.astype(_bf) for w in (Wg1, Wg2, Wg3, Wg4, Wg5, Wg6, Wg7, Wg8)]

    h1b, h2b, h3b, z_ae, x_bar, m1b = _run_ae(x, wb, bb, wg[0])

    adjb, m2b = _run_cast_l1(adj, m1b, h1b)
    m3b = _spmm(_l2_body, adjb, m2b, [(wg[1], False), (h2b, True)],
                [(256, _bf)], bm=BM_WIDE)
    m4b = _spmm(_l3_body, adjb, m3b,
                [(wg[2], False), (h3b, True), (wg[3], False)], [(64, _bf)],
                bm=BM_WIDE)
    zgb, m5b = _spmm(_l4_body, adjb, m4b, [(z_ae, True)],
                     [(64, _bf), (128, _bf)], bm=BM_WIDE)
    z_l, m6b = _spmm(_l5_body, adjb, m5b, [(wg[4], False), (wg[5], False)],
                     [(64, _f32), (256, _bf)], bm=BM_WIDE)
    m7b = _spmm(_l6_body, adjb, m6b, [(wg[6], False)], [(128, _bf)],
                bm=BM_WIDE)
    m8b, adj_hat = _spmm(_l7_body, adjb, m7b, [(zgb, True), (zgb, False)],
                         [(128, _bf), (N, _f32)])
    z_hat, q, q1 = _spmm(
        _l8_body, adjb, m8b,
        [(wg[7], False), (z_l, True), (z_ae, True), (cluster, False)],
        [(512, _f32), (16, _f32), (16, _f32)], bm=BM_WIDE)
    return (x_bar, z_hat, adj_hat, z_ae, q, q1, z_l)


# speedup vs baseline: 1.4762x; 1.4762x over previous
import functools

import jax
import jax.numpy as jnp
from jax.experimental import pallas as pl
from jax.sharding import Mesh, PartitionSpec as P

N = 10000
A = 0.5
V = 1.0

BM = 400
BM_WIDE = 800
BM_CAST = 400

_bf = jnp.bfloat16
_f32 = jnp.float32


def _dot(a, b):
    return jnp.dot(a, b, preferred_element_type=_f32)


def _full(arr):
    return pl.BlockSpec(arr.shape, lambda i: (0,) * arr.ndim)


def _ae_body(x_ref,
             we1, be1, we2, be2, we3, be3, wz, bz,
             wd1, bd1, wd2, bd2, wd3, bd3, wxb, bxb, wg1,
             h1_ref, h2_ref, h3_ref, zae_ref, xbar_ref, m1_ref):
    xb = x_ref[...].astype(_bf)
    h1 = jax.nn.relu(_dot(xb, we1[...]) + be1[...])
    h1_ref[...] = h1.astype(_bf)
    h2 = jax.nn.relu(_dot(h1.astype(_bf), we2[...]) + be2[...])
    h2_ref[...] = h2.astype(_bf)
    h3 = jax.nn.relu(_dot(h2.astype(_bf), we3[...]) + be3[...])
    h3_ref[...] = h3.astype(_bf)
    zae = _dot(h3.astype(_bf), wz[...]) + bz[...]
    zae_ref[...] = zae
    dd1 = jax.nn.relu(_dot(zae.astype(_bf), wd1[...]) + bd1[...])
    dd2 = jax.nn.relu(_dot(dd1.astype(_bf), wd2[...]) + bd2[...])
    dd3 = jax.nn.relu(_dot(dd2.astype(_bf), wd3[...]) + bd3[...])
    xbar_ref[...] = _dot(dd3.astype(_bf), wxb[...]) + bxb[...]
    m1_ref[...] = _dot(xb, wg1[...]).astype(_bf)


def _run_ae(x, wb, bb, wg1b):
    nr = x.shape[0]
    row = lambda c: pl.BlockSpec((BM, c), lambda i: (i, 0))
    in_specs = [row(512)]
    args = [x]
    for w, b in zip(wb, bb):
        in_specs += [_full(w), _full(b)]
        args += [w, b]
    in_specs.append(_full(wg1b))
    args.append(wg1b)
    out_shape = [
        jax.ShapeDtypeStruct((nr, 128), _bf),
        jax.ShapeDtypeStruct((nr, 256), _bf),
        jax.ShapeDtypeStruct((nr, 512), _bf),
        jax.ShapeDtypeStruct((nr, 64), _f32),
        jax.ShapeDtypeStruct((nr, 512), _f32),
        jax.ShapeDtypeStruct((nr, 128), _bf),
    ]
    out_specs = [row(128), row(256), row(512), row(64), row(512), row(128)]
    return pl.pallas_call(
        _ae_body, grid=(pl.cdiv(nr, BM),), in_specs=in_specs,
        out_specs=out_specs, out_shape=out_shape)(*args)


def _cast_l1_body(adj_ref, m1_ref, h1_ref, adjb_ref, m2_ref):
    ab = adj_ref[...].astype(_bf)
    adjb_ref[...] = ab
    ge1 = jax.nn.relu(_dot(ab, m1_ref[...]))
    m2_ref[...] = ((1.0 - A) * ge1
                   + A * h1_ref[...].astype(_f32)).astype(_bf)


def _run_cast_l1(adj, m1b, h1b):
    nr = adj.shape[0]
    return pl.pallas_call(
        _cast_l1_body, grid=(pl.cdiv(nr, BM_CAST),),
        in_specs=[
            pl.BlockSpec((BM_CAST, N), lambda i: (i, 0)),
            pl.BlockSpec((N, 128), lambda i: (0, 0)),
            pl.BlockSpec((BM_CAST, 128), lambda i: (i, 0)),
        ],
        out_specs=[
            pl.BlockSpec((BM_CAST, N), lambda i: (i, 0)),
            pl.BlockSpec((BM_CAST, 128), lambda i: (i, 0)),
        ],
        out_shape=[
            jax.ShapeDtypeStruct((nr, N), _bf),
            jax.ShapeDtypeStruct((nr, 128), _bf),
        ])(adj, m1b, h1b)


def _l2_body(adj_ref, m_ref, w_ref, h_ref, out_ref):
    acc = _dot(adj_ref[...], m_ref[...]).astype(_bf)
    ge = jax.nn.relu(_dot(acc, w_ref[...]))
    out_ref[...] = ((1.0 - A) * ge + A * h_ref[...].astype(_f32)).astype(_bf)


def _l3_body(adj_ref, m_ref, w_ref, h_ref, w2_ref, out_ref):
    acc = _dot(adj_ref[...], m_ref[...]).astype(_bf)
    ge = jax.nn.relu(_dot(acc, w_ref[...]))
    mix = ((1.0 - A) * ge + A * h_ref[...].astype(_f32)).astype(_bf)
    out_ref[...] = _dot(mix, w2_ref[...]).astype(_bf)


def _l4_body(adj_ref, m_ref, zae_ref, zgb_ref, m5_ref):
    zg = _dot(adj_ref[...], m_ref[...])
    zgb = zg.astype(_bf)
    zgb_ref[...] = zgb
    zi = ((1.0 - A) * zg + A * zae_ref[...]).astype(_bf)
    m5_ref[...] = jnp.concatenate([zi, zgb], axis=1)


def _l5_body(adj_ref, m_ref, w5_ref, w6_ref, zl_ref, m6_ref):
    acc = _dot(adj_ref[...], m_ref[...])
    zl_ref[...] = acc[:, :64]
    gd1 = jax.nn.relu(_dot(acc[:, 64:].astype(_bf), w5_ref[...])).astype(_bf)
    m6_ref[...] = _dot(gd1, w6_ref[...]).astype(_bf)


def _l6_body(adj_ref, m_ref, w_ref, out_ref):
    gd2 = jax.nn.relu(_dot(adj_ref[...], m_ref[...])).astype(_bf)
    out_ref[...] = _dot(gd2, w_ref[...]).astype(_bf)


def _l7_body(adj_ref, m_ref, zgb_ref, zgfull_ref, out_ref, ah_ref):
    out_ref[...] = jax.nn.relu(_dot(adj_ref[...], m_ref[...])).astype(_bf)
    prod = jax.lax.dot_general(
        zgb_ref[...], zgfull_ref[...], (((1,), (1,)), ((), ())),
        preferred_element_type=_f32)
    ah_ref[...] = jax.nn.sigmoid(prod)


def _softq(z, c, cc, q_ref):
    zz = jnp.sum(z * z, axis=1, keepdims=True)
    zc = jax.lax.dot_general(z, c, (((1,), (1,)), ((), ())),
                             preferred_element_type=_f32,
                             precision=jax.lax.Precision.HIGHEST)
    d2 = zz + cc - 2.0 * zc
    q = 1.0 / (1.0 + d2 / V)
    q = q ** ((V + 1.0) / 2.0)
    q_ref[...] = q / jnp.sum(q, axis=1, keepdims=True)


def _l8_body(adj_ref, m_ref, w_ref, zl_ref, zae_ref, c_ref,
             out_ref, q_ref, q1_ref):
    acc = _dot(adj_ref[...], m_ref[...]).astype(_bf)
    out_ref[...] = jax.nn.relu(_dot(acc, w_ref[...]))
    c = c_ref[...]
    cc = jnp.sum(c * c, axis=1)[None, :]
    _softq(zl_ref[...], c, cc, q_ref)
    _softq(zae_ref[...], c, cc, q1_ref)


def _spmm(body, adjb, m, extras, outs, bm=BM):
    nr = adjb.shape[0]
    row = lambda c: pl.BlockSpec((bm, c), lambda i: (i, 0))
    in_specs = [row(N), _full(m)]
    args = [adjb, m]
    for e, blocked in extras:
        in_specs.append(row(e.shape[1]) if blocked else _full(e))
        args.append(e)
    out_specs = [row(c) for c, _ in outs]
    out_shape = [jax.ShapeDtypeStruct((nr, c), dt) for c, dt in outs]
    if len(outs) == 1:
        out_specs, out_shape = out_specs[0], out_shape[0]
    return pl.pallas_call(
        body, grid=(pl.cdiv(nr, bm),), in_specs=in_specs,
        out_specs=out_specs, out_shape=out_shape)(*args)


def _pipeline(x, adj, wb, bb, wg, cluster, axis=None):
    ag = (lambda t: jax.lax.all_gather(t, axis, axis=0, tiled=True)) \
        if axis else (lambda t: t)

    h1b, h2b, h3b, z_ae, x_bar, m1b = _run_ae(x, wb, bb, wg[0])

    adjb, m2b = _run_cast_l1(adj, ag(m1b), h1b)
    m3b = _spmm(_l2_body, adjb, ag(m2b), [(wg[1], False), (h2b, True)],
                [(256, _bf)], bm=BM_WIDE)
    m4b = _spmm(_l3_body, adjb, ag(m3b),
                [(wg[2], False), (h3b, True), (wg[3], False)], [(64, _bf)],
                bm=BM_WIDE)
    zgb, m5b = _spmm(_l4_body, adjb, ag(m4b), [(z_ae, True)],
                     [(64, _bf), (128, _bf)], bm=BM_WIDE)
    z_l, m6b = _spmm(_l5_body, adjb, ag(m5b),
                     [(wg[4], False), (wg[5], False)],
                     [(64, _f32), (256, _bf)], bm=BM_WIDE)
    m7b = _spmm(_l6_body, adjb, ag(m6b), [(wg[6], False)], [(128, _bf)],
                bm=BM_WIDE)
    m8b, adj_hat = _spmm(_l7_body, adjb, ag(m7b),
                         [(zgb, True), (ag(zgb), False)],
                         [(128, _bf), (N, _f32)])
    z_hat, q, q1 = _spmm(
        _l8_body, adjb, ag(m8b),
        [(wg[7], False), (z_l, True), (z_ae, True), (cluster, False)],
        [(512, _f32), (16, _f32), (16, _f32)], bm=BM_WIDE)
    return x_bar, z_hat, adj_hat, z_ae, q, q1, z_l


def kernel(x, adj, We1, be1, We2, be2, We3, be3, Wz, bz, Wd1, bd1, Wd2, bd2,
           Wd3, bd3, Wxb, bxb, Wg1, Wg2, Wg3, Wg4, Wg5, Wg6, Wg7, Wg8,
           cluster):
    wb = [w.astype(_bf) for w in (We1, We2, We3, Wz, Wd1, Wd2, Wd3, Wxb)]
    bb = [b.reshape(1, -1) for b in (be1, be2, be3, bz, bd1, bd2, bd3, bxb)]
    wg = [w.astype(_bf) for w in (Wg1, Wg2, Wg3, Wg4, Wg5, Wg6, Wg7, Wg8)]

    devs = jax.devices()
    nshards = 2 if len(devs) >= 2 else 1
    if nshards == 1:
        return _pipeline(x, adj, wb, bb, wg, cluster)

    mesh = Mesh(devs[:nshards], ("r",))
    rowsharded = P("r", None)
    rep2 = P(None, None)
    fn = jax.shard_map(
        functools.partial(_pipeline, axis="r"),
        mesh=mesh,
        in_specs=(rowsharded, rowsharded, [rep2] * 8, [rep2] * 8,
                  [rep2] * 8, rep2),
        out_specs=(rowsharded,) * 7,
        check_vma=False,
    )
    return fn(x, adj, wb, bb, wg, cluster)
